# baseline (speedup 1.0000x reference)
import functools

import jax
import jax.numpy as jnp
from jax.experimental import pallas as pl
from jax.experimental.pallas import tpu as pltpu

_MODULE_NUM = 8
_OFFS = [0, 1, 3, 6, 10, 15, 21, 28, 36]
_GOUT = 36
_BLK = 128


def _dt(a, b):
    return jax.lax.dot_general(
        a, b, (((1,), (1,)), ((), ())), preferred_element_type=jnp.float32
    )


def _mm(a, b):
    return jax.lax.dot_general(
        a, b, (((1,), (0,)), ((), ())), preferred_element_type=jnp.float32
    )


def _routing(gl):
    col = jax.lax.broadcasted_iota(jnp.int32, (1, _GOUT), 1)
    neg = jnp.float32(-1e30)
    gates = jnp.zeros_like(gl)
    onehot = jnp.zeros_like(gl)
    softm = jnp.zeros_like(gl)
    for i in range(_MODULE_NUM):
        lo, hi = _OFFS[i], _OFFS[i + 1]
        m = hi - lo
        mask = jnp.logical_and(col >= lo, col < hi)
        maskf = mask.astype(jnp.float32)
        if m == 1:
            gates = gates + maskf
            onehot = onehot + maskf
            softm = softm + maskf
            continue
        glm = jnp.where(mask, gl, neg)
        m1 = jnp.max(glm, axis=-1, keepdims=True)
        is1 = glm == m1
        idx1 = jnp.min(jnp.where(is1, col, _GOUT * 2), axis=-1, keepdims=True)
        oh1 = (col == idx1).astype(jnp.float32)
        glm2 = jnp.where(col == idx1, neg, glm)
        m2 = jnp.max(glm2, axis=-1, keepdims=True)
        is2 = glm2 == m2
        idx2 = jnp.min(jnp.where(is2, col, _GOUT * 2), axis=-1, keepdims=True)
        oh2 = (col == idx2).astype(jnp.float32)
        p1 = jax.nn.sigmoid(m1 - m2)
        gates = gates + oh1 * p1 + oh2 * (1.0 - p1)
        onehot = onehot + oh1 + oh2
        e = jnp.where(mask, jnp.exp(glm - m1), 0.0)
        s = jnp.sum(e, axis=-1, keepdims=True)
        softm = softm + e / s
    return gates, onehot, softm


def _kern(x_ref, gi_ref, w0_ref, b0_ref, wr_ref, br_ref, g0_ref, g0b_ref,
          g1_ref, g1b_ref, out_ref, gates_ref, oh_ref, sm_ref, o_scr):
    blk = _BLK
    gi = gi_ref[...]
    h = jnp.maximum(_dt(gi, g0_ref[...]) + g0b_ref[...], 0.0)
    gl = _dt(h, g1_ref[...]) + g1b_ref[...]
    gates, onehot, softm = _routing(gl)
    gates_ref[...] = gates
    oh_ref[...] = onehot
    sm_ref[...] = softm

    li = jax.lax.broadcasted_iota(jnp.int32, (blk, _MODULE_NUM * blk), 1)
    ti = jax.lax.broadcasted_iota(jnp.int32, (blk, _MODULE_NUM * blk), 0)
    diag = (li & (blk - 1)) == ti

    def selector(stage):
        n_in = stage + 1
        off = _OFFS[stage]
        r = jnp.concatenate(
            [jnp.broadcast_to(gates[:, off + j:off + j + 1], (blk, blk))
             for j in range(n_in)], axis=1)
        return jnp.where(diag[:, :n_in * blk], r, 0.0).astype(jnp.bfloat16)

    x = x_ref[...]
    out0 = jnp.maximum(_dt(x, w0_ref[...]) + b0_ref[...], 0.0)
    o_scr[0:blk, :] = out0.astype(jnp.bfloat16)
    for i in range(_MODULE_NUM - 1):
        n_in = i + 1
        fc_in = _mm(selector(i), o_scr[0:n_in * blk, :])
        fc = jnp.maximum(
            _dt(fc_in.astype(jnp.bfloat16), wr_ref[i]) + br_ref[i], 0.0) + fc_in
        o_scr[n_in * blk:(n_in + 1) * blk, :] = fc.astype(jnp.bfloat16)
    out_ref[...] = _mm(selector(_MODULE_NUM - 1), o_scr[...])


@functools.partial(jax.jit, static_argnames=())
def kernel(module_input, gate_input, W0, b0, W_rest, b_rest, G0_w, G0_b, G1_w, G1_b):
    B, D = module_input.shape
    H = W0.shape[0]
    blk = _BLK
    grid = (B // blk,)

    def row_map(i):
        return (i, 0)

    def const_map2(i):
        return (0, 0)

    def const_map1(i):
        return (0,)

    def const_map3(i):
        return (0, 0, 0)

    out_shapes = (
        jax.ShapeDtypeStruct((B, H), jnp.float32),
        jax.ShapeDtypeStruct((B, _GOUT), jnp.float32),
        jax.ShapeDtypeStruct((B, _GOUT), jnp.float32),
        jax.ShapeDtypeStruct((B, _GOUT), jnp.float32),
    )
    in_specs = [
        pl.BlockSpec((blk, D), row_map),
        pl.BlockSpec((blk, gate_input.shape[1]), row_map),
        pl.BlockSpec(W0.shape, const_map2),
        pl.BlockSpec(b0.shape, const_map1),
        pl.BlockSpec(W_rest.shape, const_map3),
        pl.BlockSpec(b_rest.shape, const_map2),
        pl.BlockSpec(G0_w.shape, const_map2),
        pl.BlockSpec(G0_b.shape, const_map1),
        pl.BlockSpec(G1_w.shape, const_map2),
        pl.BlockSpec(G1_b.shape, const_map1),
    ]
    out_specs = (
        pl.BlockSpec((blk, H), row_map),
        pl.BlockSpec((blk, _GOUT), row_map),
        pl.BlockSpec((blk, _GOUT), row_map),
        pl.BlockSpec((blk, _GOUT), row_map),
    )
    return pl.pallas_call(
        _kern,
        grid=grid,
        in_specs=in_specs,
        out_specs=out_specs,
        out_shape=out_shapes,
        scratch_shapes=[pltpu.VMEM((_MODULE_NUM * blk, H), jnp.bfloat16)],
    )(module_input.astype(jnp.bfloat16), gate_input,
      W0.astype(jnp.bfloat16), b0, W_rest.astype(jnp.bfloat16), b_rest,
      G0_w, G0_b, G1_w, G1_b)


# speedup vs baseline: 1.8706x; 1.8706x over previous
import functools

import jax
import jax.numpy as jnp
from jax.experimental import pallas as pl
from jax.experimental.pallas import tpu as pltpu

_MODULE_NUM = 8
_OFFS = [0, 1, 3, 6, 10, 15, 21, 28, 36]
_GOUT = 36
_GBLK = 1024
_DBLK = 512


def _dt(a, b):
    return jax.lax.dot_general(
        a, b, (((1,), (1,)), ((), ())), preferred_element_type=jnp.float32
    )


def _routing(gl):
    col = jax.lax.broadcasted_iota(jnp.int32, (1, _GOUT), 1)
    neg = jnp.float32(-1e30)
    gates = jnp.zeros_like(gl)
    onehot = jnp.zeros_like(gl)
    softm = jnp.zeros_like(gl)
    for i in range(_MODULE_NUM):
        lo, hi = _OFFS[i], _OFFS[i + 1]
        m = hi - lo
        mask = jnp.logical_and(col >= lo, col < hi)
        maskf = mask.astype(jnp.float32)
        if m == 1:
            gates = gates + maskf
            onehot = onehot + maskf
            softm = softm + maskf
            continue
        glm = jnp.where(mask, gl, neg)
        m1 = jnp.max(glm, axis=-1, keepdims=True)
        is1 = glm == m1
        idx1 = jnp.min(jnp.where(is1, col, _GOUT * 2), axis=-1, keepdims=True)
        oh1 = (col == idx1).astype(jnp.float32)
        glm2 = jnp.where(col == idx1, neg, glm)
        m2 = jnp.max(glm2, axis=-1, keepdims=True)
        is2 = glm2 == m2
        idx2 = jnp.min(jnp.where(is2, col, _GOUT * 2), axis=-1, keepdims=True)
        oh2 = (col == idx2).astype(jnp.float32)
        p1 = jax.nn.sigmoid(m1 - m2)
        gates = gates + oh1 * p1 + oh2 * (1.0 - p1)
        onehot = onehot + oh1 + oh2
        e = jnp.where(mask, jnp.exp(glm - m1), 0.0)
        s = jnp.sum(e, axis=-1, keepdims=True)
        softm = softm + e / s
    return gates, onehot, softm


def _gkern(gi_ref, g0_ref, g0b_ref, g1_ref, g1b_ref,
           gates_ref, oh_ref, sm_ref):
    gi = gi_ref[...]
    h = jnp.maximum(_dt(gi, g0_ref[...]) + g0b_ref[...], 0.0)
    gl = _dt(h, g1_ref[...]) + g1b_ref[...]
    gates, onehot, softm = _routing(gl)
    gates_ref[...] = gates
    oh_ref[...] = onehot
    sm_ref[...] = softm


def _dkern(x_ref, g_ref, w0_ref, b0_ref, wr_ref, br_ref, out_ref):
    gates = g_ref[...]
    x = x_ref[...]
    outs = [jnp.maximum(_dt(x, w0_ref[...]) + b0_ref[...], 0.0)]
    for i in range(_MODULE_NUM - 1):
        fc_in = outs[0] * gates[:, _OFFS[i]:_OFFS[i] + 1]
        for j in range(1, i + 1):
            fc_in = fc_in + outs[j] * gates[:, _OFFS[i] + j:_OFFS[i] + j + 1]
        fc = jnp.maximum(_dt(fc_in, wr_ref[i]) + br_ref[i], 0.0) + fc_in
        outs.append(fc)
    last = outs[0] * gates[:, _OFFS[7]:_OFFS[7] + 1]
    for j in range(1, _MODULE_NUM):
        last = last + outs[j] * gates[:, _OFFS[7] + j:_OFFS[7] + j + 1]
    out_ref[...] = last


def _row2(i):
    return (i, 0)


def _c1(i):
    return (0,)


def _c2(i):
    return (0, 0)


def _c3(i):
    return (0, 0, 0)


@functools.partial(jax.jit, static_argnames=())
def kernel(module_input, gate_input, W0, b0, W_rest, b_rest, G0_w, G0_b, G1_w, G1_b):
    B, D = module_input.shape
    H = W0.shape[0]

    gates, onehot, softm = pl.pallas_call(
        _gkern,
        grid=(B // _GBLK,),
        in_specs=[
            pl.BlockSpec((_GBLK, gate_input.shape[1]), _row2),
            pl.BlockSpec(G0_w.shape, _c2),
            pl.BlockSpec(G0_b.shape, _c1),
            pl.BlockSpec(G1_w.shape, _c2),
            pl.BlockSpec(G1_b.shape, _c1),
        ],
        out_specs=(
            pl.BlockSpec((_GBLK, _GOUT), _row2),
            pl.BlockSpec((_GBLK, _GOUT), _row2),
            pl.BlockSpec((_GBLK, _GOUT), _row2),
        ),
        out_shape=(
            jax.ShapeDtypeStruct((B, _GOUT), jnp.float32),
            jax.ShapeDtypeStruct((B, _GOUT), jnp.float32),
            jax.ShapeDtypeStruct((B, _GOUT), jnp.float32),
        ),
    )(gate_input, G0_w, G0_b, G1_w, G1_b)

    last = pl.pallas_call(
        _dkern,
        grid=(B // _DBLK,),
        in_specs=[
            pl.BlockSpec((_DBLK, D), _row2),
            pl.BlockSpec((_DBLK, _GOUT), _row2),
            pl.BlockSpec(W0.shape, _c2),
            pl.BlockSpec(b0.shape, _c1),
            pl.BlockSpec(W_rest.shape, _c3),
            pl.BlockSpec(b_rest.shape, _c2),
        ],
        out_specs=pl.BlockSpec((_DBLK, H), _row2),
        out_shape=jax.ShapeDtypeStruct((B, H), jnp.float32),
        compiler_params=pltpu.CompilerParams(vmem_limit_bytes=66_000_000),
    )(module_input, gates, W0, b0, W_rest, b_rest)

    return last, gates, onehot, softm
